# Initial kernel scaffold; baseline (speedup 1.0000x reference)
import jax
import jax.numpy as jnp
from jax.experimental import pallas as pl


def kernel(x, tables, lin_w, lin_b):
    raise NotImplementedError("write your pallas kernel here")


# speedup vs baseline: 1.4758x; 1.4758x over previous
import functools

import numpy as np
import jax
import jax.numpy as jnp
from jax import lax
from jax.experimental import pallas as pl
from jax.experimental.pallas import tpu as pltpu
from jax.experimental.pallas import tpu_sc as plsc

_FEATURE_DIMS = (4000,) * 26
_F = 26
_V = sum(_FEATURE_DIMS) + 1
_D = 16
_B = 4096
_P = (_F * (_F - 1)) // 2
_OFFSETS = np.concatenate([[0], np.cumsum(_FEATURE_DIMS)[:-1]]).astype(np.int32)
_PI, _PJ = np.triu_indices(_F, k=1)

_NC, _NS = 2, 16
_NW = _NC * _NS
_BPW = _B // _NW
_CB = 4
_NCHUNK = _BPW // _CB
_RPB = 672
_GRP = (_CB * _RPB) // 128
_LINW = 32

_mesh = plsc.VectorSubcoreMesh(core_axis_name="c", subcore_axis_name="s")


def _lane_sum(v):
    for sh in (8, 4, 2, 1):
        perm = lax.iota(jnp.int32, _D) ^ sh
        v = v + v.at[perm].get(mode="promise_in_bounds")
    return v


@functools.partial(
    pl.kernel,
    mesh=_mesh,
    compiler_params=pltpu.CompilerParams(use_tc_tiling_on_sc=False),
    out_type=jax.ShapeDtypeStruct((_B,), jnp.float32),
    scratch_types=[
        pltpu.VMEM((_CB * _RPB,), jnp.int32),
        pltpu.VMEM((_CB * _RPB, _D), jnp.float32),
        pltpu.VMEM((_CB * _LINW,), jnp.int32),
        pltpu.VMEM((_CB * _LINW, _D), jnp.float32),
        pltpu.VMEM((_BPW,), jnp.float32),
        pltpu.SemaphoreType.DMA,
        pltpu.SemaphoreType.DMA,
    ],
)
def _ffm_sc(tab, lin_tab, idx_hbm, lidx_hbm, out_hbm,
            idx_v, rows_v, lidx_v, lrows_v, out_v, sem, lsem):
    cid = lax.axis_index("c")
    sid = lax.axis_index("s")
    wid = sid * _NC + cid
    b0 = wid * _BPW

    def group(g, carry):
        res = jnp.zeros((_D,), jnp.float32)
        for u in range(_D // _CB):
            c = g * (_D // _CB) + u
            pltpu.sync_copy(
                idx_hbm.at[pl.ds((b0 + c * _CB) * _RPB, _CB * _RPB)], idx_v)
            pltpu.sync_copy(
                lidx_hbm.at[pl.ds((b0 + c * _CB) * _LINW, _CB * _LINW)],
                lidx_v)
            copies = [
                pltpu.async_copy(tab.at[idx_v.at[pl.ds(k * 128, 128)]],
                                 rows_v.at[pl.ds(k * 128, 128)], sem)
                for k in range(_GRP)
            ]
            lcopy = pltpu.async_copy(lin_tab.at[lidx_v], lrows_v, lsem)
            for cp in copies:
                cp.wait()
            lcopy.wait()

            for bl in range(_CB):
                base = bl * _RPB

                def pair_step(p, acc):
                    a = rows_v[base + 2 * p]
                    b = rows_v[base + 2 * p + 1]
                    return acc + a * b

                acc = lax.fori_loop(0, _P, pair_step,
                                    jnp.zeros((_D,), jnp.float32), unroll=5)

                lbase = bl * _LINW

                def lin_step(j, acc):
                    return acc + lrows_v[lbase + j]

                acc = lax.fori_loop(0, _F + 1, lin_step, acc, unroll=9)
                zvec = _lane_sum(acc)
                lane = u * _CB + bl
                res = jnp.where(lax.iota(jnp.int32, _D) == lane, zvec, res)
        out_v[pl.ds(pl.multiple_of(g * _D, _D), _D)] = res
        return carry

    lax.fori_loop(0, _BPW // _D, group, 0)

    for g in range(_BPW // _D):
        z = out_v[pl.ds(g * _D, _D)]
        out_v[pl.ds(g * _D, _D)] = 1.0 / (1.0 + jnp.exp(-z))
    pltpu.sync_copy(out_v, out_hbm.at[pl.ds(b0, _BPW)])


def kernel(x, tables, lin_w, lin_b):
    x = x.astype(jnp.int32)
    adj = x + jnp.asarray(_OFFSETS)[None, :]

    col_a = adj[:, _PJ] + jnp.asarray(_PI.astype(np.int32) * _V)[None, :]
    col_b = adj[:, _PI] + jnp.asarray(_PJ.astype(np.int32) * _V)[None, :]
    pidx = jnp.stack([col_a, col_b], axis=-1).reshape(_B, 2 * _P)
    pidx = jnp.pad(pidx, ((0, 0), (0, _RPB - 2 * _P)))
    idx_flat = pidx.reshape(-1)

    lidx = jnp.concatenate(
        [adj,
         jnp.full((_B, 1), _V, jnp.int32),
         jnp.full((_B, _LINW - _F - 1), _V + 1, jnp.int32)], axis=1)
    lidx_flat = lidx.reshape(-1)

    tab_flat = tables.reshape(_F * _V, _D)
    lin_tab = jnp.pad(
        jnp.concatenate(
            [lin_w, lin_b.reshape(1, 1), jnp.zeros((1, 1), jnp.float32)],
            axis=0),
        ((0, 0), (0, _D - 1)))

    return _ffm_sc(tab_flat, lin_tab, idx_flat, lidx_flat)


# speedup vs baseline: 1.5146x; 1.0263x over previous
import functools

import numpy as np
import jax
import jax.numpy as jnp
from jax import lax
from jax.experimental import pallas as pl
from jax.experimental.pallas import tpu as pltpu
from jax.experimental.pallas import tpu_sc as plsc

_FEATURE_DIMS = (4000,) * 26
_F = 26
_FP = 32
_V = sum(_FEATURE_DIMS) + 1
_D = 16
_B = 4096
_OFFSETS = np.concatenate([[0], np.cumsum(_FEATURE_DIMS)[:-1]]).astype(np.int64)

_NC, _NS = 2, 16
_NW = _NC * _NS
_BPW = _B // _NW
_CB = 2
_NCHUNK = _BPW // _CB
_GS = 32
_RPB = _F * _GS
_GRP = (_CB * _RPB) // 128
_CPG = _D // _CB

_base = np.zeros((_RPB,), np.int64)
for _f in range(_F):
    for _t in range(_F):
        _base[_GS * _f + _t] = _t * _V + _OFFSETS[_f]
_BASE = _base.astype(np.int32)

_lbase = np.zeros((_FP,), np.int64)
_lbase[:_F] = _OFFSETS
_lbase[_F] = _V
_lbase[_F + 1:] = _V + 1
_LBASE = _lbase.astype(np.int32)

_mesh = plsc.VectorSubcoreMesh(core_axis_name="c", subcore_axis_name="s")


def _lane_sum(v):
    for sh in (8, 4, 2, 1):
        perm = lax.iota(jnp.int32, _D) ^ sh
        v = v + v.at[perm].get(mode="promise_in_bounds")
    return v


@functools.partial(
    pl.kernel,
    mesh=_mesh,
    compiler_params=pltpu.CompilerParams(use_tc_tiling_on_sc=False),
    out_type=jax.ShapeDtypeStruct((_B,), jnp.float32),
    scratch_types=[
        pltpu.VMEM((_RPB,), jnp.int32),
        pltpu.VMEM((_FP,), jnp.int32),
        pltpu.VMEM((_CB * _FP,), jnp.int32),
        pltpu.VMEM((_CB * _RPB,), jnp.int32),
        pltpu.VMEM((_CB * _RPB, _D), jnp.float32),
        pltpu.VMEM((_CB * _FP,), jnp.int32),
        pltpu.VMEM((_CB * _FP,), jnp.float32),
        pltpu.VMEM((_BPW,), jnp.float32),
        pltpu.SemaphoreType.DMA,
        pltpu.SemaphoreType.DMA,
    ],
)
def _ffm_sc(x_hbm, tab, lin_tab, base_hbm, lbase_hbm, out_hbm,
            base_v, lbase_v, xbuf, idx_v, rows_v, lidx_v, lrows_v, out_v,
            sem, lsem):
    cid = lax.axis_index("c")
    sid = lax.axis_index("s")
    wid = sid * _NC + cid
    b0 = wid * _BPW

    pltpu.sync_copy(base_hbm, base_v)
    pltpu.sync_copy(lbase_hbm, lbase_v)

    def group(g, carry):
        res = jnp.zeros((_D,), jnp.float32)
        for u in range(_CPG):
            c = g * _CPG + u
            pltpu.sync_copy(x_hbm.at[pl.ds((b0 + c * _CB) * _FP, _CB * _FP)],
                            xbuf)
            for bl in range(_CB):
                gb = bl * _RPB
                xv = [xbuf[pl.ds(bl * _FP, _D)],
                      xbuf[pl.ds(bl * _FP + _D, _D)]]
                for f in range(_F):
                    xs = xv[f // _D][f % _D]
                    idx_v[pl.ds(gb + _GS * f, _D)] = (
                        base_v[pl.ds(_GS * f, _D)] + xs)
                    idx_v[pl.ds(gb + _GS * f + _D, _D)] = (
                        base_v[pl.ds(_GS * f + _D, _D)] + xs)
                for s in range(_FP // _D):
                    lidx_v[pl.ds(bl * _FP + s * _D, _D)] = (
                        lbase_v[pl.ds(s * _D, _D)] +
                        xbuf[pl.ds(bl * _FP + s * _D, _D)])
            copies = [
                pltpu.async_copy(tab.at[idx_v.at[pl.ds(k * 128, 128)]],
                                 rows_v.at[pl.ds(k * 128, 128)], sem)
                for k in range(_GRP)
            ]
            lcopy = pltpu.async_copy(lin_tab.at[lidx_v], lrows_v, lsem)
            for cp in copies:
                cp.wait()
            lcopy.wait()

            for bl in range(_CB):
                gb = bl * _RPB

                def outer(i, acc):
                    def inner(j, acc):
                        a = rows_v[gb + _GS * j + i]
                        b = rows_v[gb + _GS * i + j]
                        return acc + a * b
                    return lax.fori_loop(i + 1, _F, inner, acc)

                acc = lax.fori_loop(0, _F - 1, outer,
                                    jnp.zeros((_D,), jnp.float32))
                acc = acc + (lrows_v[pl.ds(bl * _FP, _D)] +
                             lrows_v[pl.ds(bl * _FP + _D, _D)])
                zvec = _lane_sum(acc)
                lane = u * _CB + bl
                res = jnp.where(lax.iota(jnp.int32, _D) == lane, zvec, res)
        out_v[pl.ds(pl.multiple_of(g * _D, _D), _D)] = res
        return carry

    lax.fori_loop(0, _BPW // _D, group, 0)

    for g in range(_BPW // _D):
        z = out_v[pl.ds(g * _D, _D)]
        out_v[pl.ds(g * _D, _D)] = 1.0 / (1.0 + jnp.exp(-z))
    pltpu.sync_copy(out_v, out_hbm.at[pl.ds(b0, _BPW)])


def kernel(x, tables, lin_w, lin_b):
    x32 = jnp.pad(x.astype(jnp.int32), ((0, 0), (0, _FP - _F)))
    tab_flat = tables.reshape(_F * _V, _D)
    lin_flat = jnp.concatenate(
        [lin_w.reshape(-1), lin_b.reshape(1), jnp.zeros((1,), jnp.float32)])
    return _ffm_sc(x32.reshape(-1), tab_flat, lin_flat,
                   jnp.asarray(_BASE), jnp.asarray(_LBASE))


# speedup vs baseline: 1.5164x; 1.0012x over previous
import functools

import numpy as np
import jax
import jax.numpy as jnp
from jax import lax
from jax.experimental import pallas as pl
from jax.experimental.pallas import tpu as pltpu
from jax.experimental.pallas import tpu_sc as plsc

_FEATURE_DIMS = (4000,) * 26
_FDIM = 4000
_F = 26
_FP = 32
_V = sum(_FEATURE_DIMS) + 1
_D = 16
_B = 4096

_NC, _NS = 2, 16
_NW = _NC * _NS
_BPW = _B // _NW
_CB = 2
_GS = 32
_RPB = _F * _GS
_GRP = (_CB * _RPB) // 128
_CPG = _D // _CB

_mesh = plsc.VectorSubcoreMesh(core_axis_name="c", subcore_axis_name="s")


def _lane_sum(v):
    for sh in (8, 4, 2, 1):
        perm = lax.iota(jnp.int32, _D) ^ sh
        v = v + v.at[perm].get(mode="promise_in_bounds")
    return v


@functools.partial(
    pl.kernel,
    mesh=_mesh,
    compiler_params=pltpu.CompilerParams(use_tc_tiling_on_sc=False),
    out_type=jax.ShapeDtypeStruct((_B,), jnp.float32),
    scratch_types=[
        pltpu.VMEM((_CB * _FP,), jnp.int32),
        pltpu.VMEM((_CB * _RPB,), jnp.int32),
        pltpu.VMEM((_CB * _RPB, _D), jnp.float32),
        pltpu.VMEM((_CB * _FP,), jnp.int32),
        pltpu.VMEM((_CB * _FP,), jnp.float32),
        pltpu.VMEM((_BPW,), jnp.float32),
        pltpu.VMEM((_D,), jnp.float32),
        pltpu.SemaphoreType.DMA,
        pltpu.SemaphoreType.DMA,
    ],
)
def _ffm_sc(x_hbm, tab, lin_w, lin_b, out_hbm,
            xbuf, idx_v, rows_v, lidx_v, lrows_v, out_v, bias_v, sem, lsem):
    cid = lax.axis_index("c")
    sid = lax.axis_index("s")
    wid = sid * _NC + cid
    b0 = wid * _BPW

    pltpu.sync_copy(lin_b, bias_v.at[pl.ds(0, 1)])
    bias = bias_v[pl.ds(0, _D)][0]
    lanes = lax.iota(jnp.int32, _D)
    off_lo = _FDIM * lanes
    off_hi = jnp.where(lanes < _F - _D, _FDIM * (lanes + _D), 0)
    fmask_hi = lanes < _F - _D

    def group(g, carry):
        def chunk(u, res):
            c = g * _CPG + u
            pltpu.sync_copy(
                x_hbm.at[pl.ds(
                    pl.multiple_of((b0 + c * _CB) * _FP, _CB * _FP),
                    _CB * _FP)],
                xbuf)
            for bl in range(_CB):
                adj_lo = xbuf[pl.ds(bl * _FP, _D)] + off_lo
                adj_hi = xbuf[pl.ds(bl * _FP + _D, _D)] + off_hi
                lidx_v[pl.ds(bl * _FP, _D)] = adj_lo
                lidx_v[pl.ds(bl * _FP + _D, _D)] = adj_hi
                gb = bl * _RPB

                def build(t, carry2):
                    alo, ahi = carry2
                    o = pl.multiple_of(gb + _GS * t, _GS)
                    idx_v[pl.ds(o, _D)] = alo
                    idx_v[pl.ds(o + _D, _D)] = ahi
                    return alo + _V, ahi + _V

                lax.fori_loop(0, _F, build, (adj_lo, adj_hi))
            copies = [
                pltpu.async_copy(tab.at[idx_v.at[pl.ds(k * 128, 128)]],
                                 rows_v.at[pl.ds(k * 128, 128)], sem)
                for k in range(_GRP)
            ]
            lcopy = pltpu.async_copy(lin_w.at[lidx_v], lrows_v, lsem)
            for cp in copies:
                cp.wait()
            lcopy.wait()

            for bl in range(_CB):
                gb = bl * _RPB

                def outer(i, acc):
                    def inner(j, acc):
                        a = rows_v[gb + _GS * i + j]
                        b = rows_v[gb + _GS * j + i]
                        return acc + a * b
                    return lax.fori_loop(i + 1, _F, inner, acc)

                acc = lax.fori_loop(0, _F - 1, outer,
                                    jnp.zeros((_D,), jnp.float32))
                lv_lo = lrows_v[pl.ds(bl * _FP, _D)]
                lv_hi = lrows_v[pl.ds(bl * _FP + _D, _D)]
                acc = acc + lv_lo + jnp.where(fmask_hi, lv_hi, 0.0)
                zvec = _lane_sum(acc) + bias
                res = jnp.where(lanes == u * _CB + bl, zvec, res)
            return res

        res = lax.fori_loop(0, _CPG, chunk, jnp.zeros((_D,), jnp.float32))
        out_v[pl.ds(pl.multiple_of(g * _D, _D), _D)] = (
            1.0 / (1.0 + jnp.exp(-res)))
        return carry

    lax.fori_loop(0, _BPW // _D, group, 0)
    pltpu.sync_copy(out_v, out_hbm.at[pl.ds(b0, _BPW)])


def kernel(x, tables, lin_w, lin_b):
    x32 = jnp.pad(x.astype(jnp.int32), ((0, 0), (0, _FP - _F)))
    tab_flat = tables.reshape(_F * _V, _D)
    return _ffm_sc(x32.reshape(-1), tab_flat, lin_w.reshape(-1), lin_b)


# speedup vs baseline: 4.6195x; 3.0464x over previous
import functools

import numpy as np
import jax
import jax.numpy as jnp
from jax import lax
from jax.experimental import pallas as pl
from jax.experimental.pallas import tpu as pltpu
from jax.experimental.pallas import tpu_sc as plsc

_FEATURE_DIMS = (4000,) * 26
_FDIM = 4000
_F = 26
_FP = 32
_V = sum(_FEATURE_DIMS) + 1
_D = 16
_B = 4096
_SLOTS = 32
_QB = _SLOTS * _D // 128
_LIN_SLOT = _F
_BIAS_SLOT = _F + 1

_NC, _NS = 2, 16
_NW = _NC * _NS
_BPW = _B // _NW
_CB = 2
_BPB = _FP * _QB
_GRP = (_CB * _BPB) // 128
_CPG = _D // _CB

_mesh = plsc.VectorSubcoreMesh(core_axis_name="c", subcore_axis_name="s")


def _lane_sum(v):
    for sh in (8, 4, 2, 1):
        perm = lax.iota(jnp.int32, _D) ^ sh
        v = v + v.at[perm].get(mode="promise_in_bounds")
    return v


@functools.partial(
    pl.kernel,
    mesh=_mesh,
    compiler_params=pltpu.CompilerParams(use_tc_tiling_on_sc=False),
    out_type=jax.ShapeDtypeStruct((_B,), jnp.float32),
    scratch_types=[
        pltpu.VMEM((_CB * _FP,), jnp.int32),
        pltpu.VMEM((_CB * _BPB,), jnp.int32),
        pltpu.VMEM((_CB * _BPB, 128), jnp.float32),
        pltpu.VMEM((_BPW,), jnp.float32),
        pltpu.SemaphoreType.DMA,
    ],
)
def _ffm_sc(x_hbm, tab, out_hbm, xbuf, idx_v, rows_v, out_v, sem):
    cid = lax.axis_index("c")
    sid = lax.axis_index("s")
    wid = sid * _NC + cid
    b0 = wid * _BPW

    lanes = lax.iota(jnp.int32, _D)
    off_lo = _FDIM * lanes
    off_hi = jnp.where(lanes < _F - _D, _FDIM * (lanes + _D), 0)

    def group(g, carry):
        def chunk(u, res):
            c = g * _CPG + u
            pltpu.sync_copy(
                x_hbm.at[pl.ds(
                    pl.multiple_of((b0 + c * _CB) * _FP, _CB * _FP),
                    _CB * _FP)],
                xbuf)
            for bl in range(_CB):
                adj_lo = (xbuf[pl.ds(bl * _FP, _D)] + off_lo) * _QB
                adj_hi = (xbuf[pl.ds(bl * _FP + _D, _D)] + off_hi) * _QB
                for q in range(_QB):
                    idx_v[pl.ds(bl * _BPB + q * _FP, _D)] = adj_lo + q
                    idx_v[pl.ds(bl * _BPB + q * _FP + _D, _D)] = adj_hi + q
            copies = [
                pltpu.async_copy(tab.at[idx_v.at[pl.ds(k * 128, 128)]],
                                 rows_v.at[pl.ds(k * 128, 128)], sem)
                for k in range(_GRP)
            ]
            for cp in copies:
                cp.wait()

            for bl in range(_CB):
                gb = bl * _BPB

                def outer(i, acc):
                    blk_a = gb + (i >> 3) * _FP
                    sub_a = pl.multiple_of((i & 7) * _D, _D)

                    def inner(j, acc):
                        a = rows_v[blk_a + j, pl.ds(sub_a, _D)]
                        b = rows_v[gb + (j >> 3) * _FP + i,
                                   pl.ds(pl.multiple_of((j & 7) * _D, _D),
                                         _D)]
                        return acc + a * b
                    return lax.fori_loop(i + 1, _F, inner, acc)

                acc = lax.fori_loop(0, _F - 1, outer,
                                    jnp.zeros((_D,), jnp.float32))

                def lin(f, acc):
                    return acc + rows_v[gb + 3 * _FP + f,
                                        pl.ds((_LIN_SLOT % 8) * _D, _D)]

                acc = lax.fori_loop(0, _F, lin, acc)
                acc = acc + rows_v[gb + 3 * _FP,
                                   pl.ds((_BIAS_SLOT % 8) * _D, _D)]
                zvec = _lane_sum(acc)
                res = jnp.where(lanes == u * _CB + bl, zvec, res)
            return res

        res = lax.fori_loop(0, _CPG, chunk, jnp.zeros((_D,), jnp.float32))
        out_v[pl.ds(pl.multiple_of(g * _D, _D), _D)] = (
            1.0 / (1.0 + jnp.exp(-res)))
        return carry

    lax.fori_loop(0, _BPW // _D, group, 0)
    pltpu.sync_copy(out_v, out_hbm.at[pl.ds(b0, _BPW)])


def kernel(x, tables, lin_w, lin_b):
    x32 = jnp.pad(x.astype(jnp.int32), ((0, 0), (0, _FP - _F)))
    tp = jnp.transpose(tables, (1, 0, 2))
    lin_col = jnp.pad(lin_w[:, :, None], ((0, 0), (0, 0), (0, _D - 1)))
    bias_col = jnp.pad(
        jnp.broadcast_to(lin_b.reshape(1, 1, 1), (_V, 1, 1)),
        ((0, 0), (0, 0), (0, _D - 1)))
    zpad = jnp.zeros((_V, _SLOTS - _F - 2, _D), jnp.float32)
    t2 = jnp.concatenate([tp, lin_col, bias_col, zpad], axis=1)
    t2 = t2.reshape(_V * _QB, 128)
    return _ffm_sc(x32.reshape(-1), t2)


# speedup vs baseline: 4.6213x; 1.0004x over previous
import functools

import numpy as np
import jax
import jax.numpy as jnp
from jax import lax
from jax.experimental import pallas as pl
from jax.experimental.pallas import tpu as pltpu
from jax.experimental.pallas import tpu_sc as plsc

_FEATURE_DIMS = (4000,) * 26
_FDIM = 4000
_F = 26
_FP = 32
_V = sum(_FEATURE_DIMS) + 1
_D = 16
_B = 4096
_SLOTS = 32
_QB = _SLOTS * _D // 128
_LIN_SLOT = _F
_BIAS_SLOT = _F + 1

_NC, _NS = 2, 16
_NW = _NC * _NS
_BPW = _B // _NW
_CB = 2
_BPB = _FP * _QB
_GRP = (_CB * _BPB) // 128
_CPG = _D // _CB

_mesh = plsc.VectorSubcoreMesh(core_axis_name="c", subcore_axis_name="s")


def _lane_sum(v):
    for sh in (8, 4, 2, 1):
        perm = lax.iota(jnp.int32, _D) ^ sh
        v = v + v.at[perm].get(mode="promise_in_bounds")
    return v


@functools.partial(
    pl.kernel,
    mesh=_mesh,
    compiler_params=pltpu.CompilerParams(use_tc_tiling_on_sc=True),
    out_type=jax.ShapeDtypeStruct((_B,), jnp.float32),
    scratch_types=[
        pltpu.VMEM((_CB * _FP,), jnp.int32),
        pltpu.VMEM((_CB * _BPB,), jnp.int32),
        pltpu.VMEM((_CB * _BPB, 128), jnp.float32),
        pltpu.VMEM((_BPW,), jnp.float32),
        pltpu.SemaphoreType.DMA,
    ],
)
def _ffm_sc(x_hbm, tab, out_hbm, xbuf, idx_v, rows_v, out_v, sem):
    cid = lax.axis_index("c")
    sid = lax.axis_index("s")
    wid = sid * _NC + cid
    b0 = wid * _BPW

    lanes = lax.iota(jnp.int32, _D)
    off_lo = _FDIM * lanes
    off_hi = jnp.where(lanes < _F - _D, _FDIM * (lanes + _D), 0)

    def group(g, carry):
        def chunk(u, res):
            c = g * _CPG + u
            pltpu.sync_copy(
                x_hbm.at[pl.ds(
                    pl.multiple_of((b0 + c * _CB) * _FP, _CB * _FP),
                    _CB * _FP)],
                xbuf)
            for bl in range(_CB):
                adj_lo = (xbuf[pl.ds(bl * _FP, _D)] + off_lo) * _QB
                adj_hi = (xbuf[pl.ds(bl * _FP + _D, _D)] + off_hi) * _QB
                for q in range(_QB):
                    idx_v[pl.ds(bl * _BPB + q * _FP, _D)] = adj_lo + q
                    idx_v[pl.ds(bl * _BPB + q * _FP + _D, _D)] = adj_hi + q
            copies = [
                pltpu.async_copy(tab.at[idx_v.at[pl.ds(k * 128, 128)]],
                                 rows_v.at[pl.ds(k * 128, 128)], sem)
                for k in range(_GRP)
            ]
            for cp in copies:
                cp.wait()

            for bl in range(_CB):
                gb = bl * _BPB

                def outer(i, acc):
                    blk_a = gb + (i >> 3) * _FP
                    sub_a = pl.multiple_of((i & 7) * _D, _D)

                    def inner(j, acc):
                        a = rows_v[blk_a + j, pl.ds(sub_a, _D)]
                        b = rows_v[gb + (j >> 3) * _FP + i,
                                   pl.ds(pl.multiple_of((j & 7) * _D, _D),
                                         _D)]
                        return acc + a * b
                    return lax.fori_loop(i + 1, _F, inner, acc)

                acc = lax.fori_loop(0, _F - 1, outer,
                                    jnp.zeros((_D,), jnp.float32))

                def lin(f, acc):
                    return acc + rows_v[gb + 3 * _FP + f,
                                        pl.ds((_LIN_SLOT % 8) * _D, _D)]

                acc = lax.fori_loop(0, _F, lin, acc)
                acc = acc + rows_v[gb + 3 * _FP,
                                   pl.ds((_BIAS_SLOT % 8) * _D, _D)]
                zvec = _lane_sum(acc)
                res = jnp.where(lanes == u * _CB + bl, zvec, res)
            return res

        res = lax.fori_loop(0, _CPG, chunk, jnp.zeros((_D,), jnp.float32))
        out_v[pl.ds(pl.multiple_of(g * _D, _D), _D)] = (
            1.0 / (1.0 + jnp.exp(-res)))
        return carry

    lax.fori_loop(0, _BPW // _D, group, 0)
    pltpu.sync_copy(out_v, out_hbm.at[pl.ds(b0, _BPW)])


def kernel(x, tables, lin_w, lin_b):
    x32 = jnp.pad(x.astype(jnp.int32), ((0, 0), (0, _FP - _F)))
    tp = jnp.transpose(tables, (1, 0, 2))
    lin_col = jnp.pad(lin_w[:, :, None], ((0, 0), (0, 0), (0, _D - 1)))
    bias_col = jnp.pad(
        jnp.broadcast_to(lin_b.reshape(1, 1, 1), (_V, 1, 1)),
        ((0, 0), (0, 0), (0, _D - 1)))
    zpad = jnp.zeros((_V, _SLOTS - _F - 2, _D), jnp.float32)
    t2 = jnp.concatenate([tp, lin_col, bias_col, zpad], axis=1)
    t2 = t2.reshape(_V * _QB, 128)
    return _ffm_sc(x32.reshape(-1), t2)


# speedup vs baseline: 5.3650x; 1.1609x over previous
import functools

import numpy as np
import jax
import jax.numpy as jnp
from jax import lax
from jax.experimental import pallas as pl
from jax.experimental.pallas import tpu as pltpu
from jax.experimental.pallas import tpu_sc as plsc

_FEATURE_DIMS = (4000,) * 26
_FDIM = 4000
_F = 26
_FP = 32
_V = sum(_FEATURE_DIMS) + 1
_D = 16
_B = 4096
_SLOTS = 32
_QB = _SLOTS * _D // 128
_LIN_SLOT = _F
_BIAS_SLOT = _F + 1

_NC, _NS = 2, 16
_NW = _NC * _NS
_BPW = _B // _NW
_CB = 2
_BPB = _FP * _QB
_GRP = (_CB * _BPB) // 128
_CPG = _D // _CB

_mesh = plsc.VectorSubcoreMesh(core_axis_name="c", subcore_axis_name="s")


def _lane_sum(v):
    for sh in (8, 4, 2, 1):
        perm = lax.iota(jnp.int32, _D) ^ sh
        v = v + v.at[perm].get(mode="promise_in_bounds")
    return v


@functools.partial(
    pl.kernel,
    mesh=_mesh,
    compiler_params=pltpu.CompilerParams(use_tc_tiling_on_sc=True),
    out_type=jax.ShapeDtypeStruct((_B,), jnp.float32),
    scratch_types=[
        pltpu.VMEM((_CB * _FP,), jnp.int32),
        pltpu.VMEM((_CB * _BPB,), jnp.int32),
        pltpu.VMEM((_CB * _BPB, 128), jnp.float32),
        pltpu.VMEM((_BPW,), jnp.float32),
        pltpu.SemaphoreType.DMA,
    ],
)
def _ffm_sc(x_hbm, tab, out_hbm, xbuf, idx_v, rows_v, out_v, sem):
    cid = lax.axis_index("c")
    sid = lax.axis_index("s")
    wid = sid * _NC + cid
    b0 = wid * _BPW

    lanes = lax.iota(jnp.int32, _D)
    off_lo = _FDIM * lanes
    off_hi = jnp.where(lanes < _F - _D, _FDIM * (lanes + _D), 0)

    def group(g, carry):
        def chunk(u, res):
            c = g * _CPG + u
            pltpu.sync_copy(
                x_hbm.at[pl.ds(
                    pl.multiple_of((b0 + c * _CB) * _FP, _CB * _FP),
                    _CB * _FP)],
                xbuf)
            for bl in range(_CB):
                adj_lo = (xbuf[pl.ds(bl * _FP, _D)] + off_lo) * _QB
                adj_hi = (xbuf[pl.ds(bl * _FP + _D, _D)] + off_hi) * _QB
                for q in range(_QB):
                    idx_v[pl.ds(bl * _BPB + q * _FP, _D)] = adj_lo + q
                    idx_v[pl.ds(bl * _BPB + q * _FP + _D, _D)] = adj_hi + q
            copies = [
                pltpu.async_copy(tab.at[idx_v.at[pl.ds(k * 128, 128)]],
                                 rows_v.at[pl.ds(k * 128, 128)], sem)
                for k in range(_GRP)
            ]
            for cp in copies:
                cp.wait()

            for bl in range(_CB):
                gb = bl * _BPB

                def outer(i, acc):
                    blk_a = gb + (i >> 3) * _FP
                    sub_a = pl.multiple_of((i & 7) * _D, _D)

                    def inner(j, acc):
                        a = rows_v[blk_a + j, pl.ds(sub_a, _D)]
                        b = rows_v[gb + (j >> 3) * _FP + i,
                                   pl.ds(pl.multiple_of((j & 7) * _D, _D),
                                         _D)]
                        return acc + a * b
                    return lax.fori_loop(i + 1, _F, inner, acc)

                acc = lax.fori_loop(0, _F - 1, outer,
                                    jnp.zeros((_D,), jnp.float32))

                def lin(f, acc):
                    return acc + rows_v[gb + 3 * _FP + f,
                                        pl.ds((_LIN_SLOT % 8) * _D, _D)]

                acc = lax.fori_loop(0, _F, lin, acc)
                acc = acc + rows_v[gb + 3 * _FP,
                                   pl.ds((_BIAS_SLOT % 8) * _D, _D)]
                zvec = _lane_sum(acc)
                res = jnp.where(lanes == u * _CB + bl, zvec, res)
            return res

        res = lax.fori_loop(0, _CPG, chunk, jnp.zeros((_D,), jnp.float32))
        out_v[pl.ds(pl.multiple_of(g * _D, _D), _D)] = (
            1.0 / (1.0 + jnp.exp(-res)))
        return carry

    lax.fori_loop(0, _BPW // _D, group, 0)
    pltpu.sync_copy(out_v, out_hbm.at[pl.ds(b0, _BPW)])


def kernel(x, tables, lin_w, lin_b):
    x32 = jnp.pad(x.astype(jnp.int32), ((0, 0), (0, _FP - _F)))
    tp = jnp.transpose(tables, (1, 0, 2)).reshape(_V, _F * _D)
    lin_col = jnp.pad(lin_w, ((0, 0), (0, _D - 1)))
    bias_col = jnp.pad(
        jnp.broadcast_to(lin_b.reshape(1, 1), (_V, 1)),
        ((0, 0), (0, _D - 1)))
    zpad = jnp.zeros((_V, (_SLOTS - _F - 2) * _D), jnp.float32)
    t2 = jnp.concatenate([tp, lin_col, bias_col, zpad], axis=1)
    t2 = t2.reshape(_V * _QB, 128)
    return _ffm_sc(x32.reshape(-1), t2)


# speedup vs baseline: 5.3664x; 1.0003x over previous
import functools

import numpy as np
import jax
import jax.numpy as jnp
from jax import lax
from jax.experimental import pallas as pl
from jax.experimental.pallas import tpu as pltpu
from jax.experimental.pallas import tpu_sc as plsc

_FEATURE_DIMS = (4000,) * 26
_FDIM = 4000
_F = 26
_FP = 32
_V = sum(_FEATURE_DIMS) + 1
_D = 16
_B = 4096
_SLOTS = 32
_QB = _SLOTS * _D // 128
_LIN_SLOT = _F
_BIAS_SLOT = _F + 1

_NC, _NS = 2, 16
_NW = _NC * _NS
_BPW = _B // _NW
_CB = 2
_BPB = _FP * _QB
_GRP = (_CB * _BPB) // 128
_CPG = _D // _CB

_mesh = plsc.VectorSubcoreMesh(core_axis_name="c", subcore_axis_name="s")


def _lane_sum(v):
    for sh in (8, 4, 2, 1):
        perm = lax.iota(jnp.int32, _D) ^ sh
        v = v + v.at[perm].get(mode="promise_in_bounds")
    return v


@functools.partial(
    pl.kernel,
    mesh=_mesh,
    compiler_params=pltpu.CompilerParams(use_tc_tiling_on_sc=True),
    out_type=jax.ShapeDtypeStruct((_B,), jnp.float32),
    scratch_types=[
        pltpu.VMEM((_CB * _FP,), jnp.int32),
        pltpu.VMEM((_CB * _BPB,), jnp.int32),
        pltpu.VMEM((_CB * _BPB, 128), jnp.float32),
        pltpu.VMEM((_BPW,), jnp.float32),
        pltpu.SemaphoreType.DMA,
    ],
)
def _ffm_sc(x_hbm, tab, out_hbm, xbuf, idx_v, rows_v, out_v, sem):
    cid = lax.axis_index("c")
    sid = lax.axis_index("s")
    wid = sid * _NC + cid
    b0 = wid * _BPW

    lanes = lax.iota(jnp.int32, _D)
    off_lo = _FDIM * lanes
    off_hi = jnp.where(lanes < _F - _D, _FDIM * (lanes + _D), 0)

    def group(g, carry):
        def chunk(u, res):
            c = g * _CPG + u
            pltpu.sync_copy(
                x_hbm.at[pl.ds(
                    pl.multiple_of((b0 + c * _CB) * _FP, _CB * _FP),
                    _CB * _FP)],
                xbuf)
            for bl in range(_CB):
                adj_lo = (xbuf[pl.ds(bl * _FP, _D)] + off_lo) * _QB
                adj_hi = (xbuf[pl.ds(bl * _FP + _D, _D)] + off_hi) * _QB
                for q in range(_QB):
                    idx_v[pl.ds(bl * _BPB + q * _FP, _D)] = adj_lo + q
                    idx_v[pl.ds(bl * _BPB + q * _FP + _D, _D)] = adj_hi + q
            copies = [
                pltpu.async_copy(tab.at[idx_v.at[pl.ds(k * 128, 128)]],
                                 rows_v.at[pl.ds(k * 128, 128)], sem)
                for k in range(_GRP)
            ]
            for cp in copies:
                cp.wait()

            for bl in range(_CB):
                gb = bl * _BPB

                zero = jnp.zeros((_D,), jnp.float32)

                def outer(i, carry):
                    a0, a1, a2, a3, dg = carry
                    blk_a = gb + (i >> 3) * _FP
                    sub_a = pl.multiple_of((i & 7) * _D, _D)
                    accs = [a0, a1, a2, a3]
                    for jq in range(_QB):
                        blk_b = gb + jq * _FP + i
                        for j8 in range(8):
                            j = 8 * jq + j8
                            if j >= _F:
                                break
                            a = rows_v[blk_a + j, pl.ds(sub_a, _D)]
                            b = rows_v[blk_b, pl.ds(j8 * _D, _D)]
                            accs[j % 4] = accs[j % 4] + a * b
                    dv = rows_v[blk_a + i, pl.ds(sub_a, _D)]
                    return (accs[0], accs[1], accs[2], accs[3],
                            dg + dv * dv)

                a0, a1, a2, a3, dg = lax.fori_loop(
                    0, _F, outer, (zero, zero, zero, zero, zero))
                acc = ((a0 + a1) + (a2 + a3) - dg) * 0.5

                def lin(f, acc):
                    return acc + rows_v[gb + 3 * _FP + f,
                                        pl.ds((_LIN_SLOT % 8) * _D, _D)]

                acc = lax.fori_loop(0, _F, lin, acc)
                acc = acc + rows_v[gb + 3 * _FP,
                                   pl.ds((_BIAS_SLOT % 8) * _D, _D)]
                zvec = _lane_sum(acc)
                res = jnp.where(lanes == u * _CB + bl, zvec, res)
            return res

        res = lax.fori_loop(0, _CPG, chunk, jnp.zeros((_D,), jnp.float32))
        out_v[pl.ds(pl.multiple_of(g * _D, _D), _D)] = (
            1.0 / (1.0 + jnp.exp(-res)))
        return carry

    lax.fori_loop(0, _BPW // _D, group, 0)
    pltpu.sync_copy(out_v, out_hbm.at[pl.ds(b0, _BPW)])


def kernel(x, tables, lin_w, lin_b):
    x32 = jnp.pad(x.astype(jnp.int32), ((0, 0), (0, _FP - _F)))
    tp = jnp.transpose(tables, (1, 0, 2)).reshape(_V, _F * _D)
    lin_col = jnp.pad(lin_w, ((0, 0), (0, _D - 1)))
    bias_col = jnp.pad(
        jnp.broadcast_to(lin_b.reshape(1, 1), (_V, 1)),
        ((0, 0), (0, _D - 1)))
    zpad = jnp.zeros((_V, (_SLOTS - _F - 2) * _D), jnp.float32)
    t2 = jnp.concatenate([tp, lin_col, bias_col, zpad], axis=1)
    t2 = t2.reshape(_V * _QB, 128)
    return _ffm_sc(x32.reshape(-1), t2)


# speedup vs baseline: 8.1160x; 1.5124x over previous
import functools

import numpy as np
import jax
import jax.numpy as jnp
from jax import lax
from jax.experimental import pallas as pl
from jax.experimental.pallas import tpu as pltpu
from jax.experimental.pallas import tpu_sc as plsc

_FEATURE_DIMS = (4000,) * 26
_FDIM = 4000
_F = 26
_FP = 32
_V = sum(_FEATURE_DIMS) + 1
_D = 16
_B = 4096
_SLOTS = 32
_QB = _SLOTS * _D // 128
_LIN_SLOT = _F
_BIAS_SLOT = _F + 1

_NC, _NS = 2, 16
_NW = _NC * _NS
_BPW = _B // _NW
_CB = 2
_BPB = _FP * _QB
_GRP = (_CB * _BPB) // 128
_CPG = _D // _CB

_mesh = plsc.VectorSubcoreMesh(core_axis_name="c", subcore_axis_name="s")


def _lane_sum(v):
    for sh in (8, 4, 2, 1):
        perm = lax.iota(jnp.int32, _D) ^ sh
        v = v + v.at[perm].get(mode="promise_in_bounds")
    return v


@functools.partial(
    pl.kernel,
    mesh=_mesh,
    compiler_params=pltpu.CompilerParams(use_tc_tiling_on_sc=True),
    out_type=jax.ShapeDtypeStruct((_B,), jnp.float32),
    scratch_types=[
        pltpu.VMEM((_CB * _FP,), jnp.int32),
        pltpu.VMEM((_CB * _BPB,), jnp.int32),
        pltpu.VMEM((_CB * _BPB, 128), jnp.float32),
        pltpu.VMEM((_BPW,), jnp.float32),
        pltpu.SemaphoreType.DMA,
    ],
)
def _ffm_sc(x_hbm, tab, out_hbm, xbuf, idx_v, rows_v, out_v, sem):
    cid = lax.axis_index("c")
    sid = lax.axis_index("s")
    wid = sid * _NC + cid
    b0 = wid * _BPW

    lanes = lax.iota(jnp.int32, _D)
    off_lo = _FDIM * lanes
    off_hi = jnp.where(lanes < _F - _D, _FDIM * (lanes + _D),
                       _FDIM * (lanes - (_F - _D)))

    def group(g, carry):
        def chunk(u, res):
            c = g * _CPG + u
            pltpu.sync_copy(
                x_hbm.at[pl.ds(
                    pl.multiple_of((b0 + c * _CB) * _FP, _CB * _FP),
                    _CB * _FP)],
                xbuf)
            for bl in range(_CB):
                adj_lo = (xbuf[pl.ds(bl * _FP, _D)] + off_lo) * _QB
                adj_hi = (xbuf[pl.ds(bl * _FP + _D, _D)] + off_hi) * _QB
                for q in range(_QB):
                    idx_v[pl.ds(bl * _BPB + q * _FP, _D)] = adj_lo + q
                    idx_v[pl.ds(bl * _BPB + q * _FP + _D, _D)] = adj_hi + q
            copies = [
                pltpu.async_copy(tab.at[idx_v.at[pl.ds(k * 128, 128)]],
                                 rows_v.at[pl.ds(k * 128, 128)], sem)
                for k in range(_GRP)
            ]
            for cp in copies:
                cp.wait()

            for bl in range(_CB):
                gb = bl * _BPB

                zero = jnp.zeros((_D,), jnp.float32)

                def outer(i, carry):
                    a0, a1, a2, a3, dg = carry
                    blk_a = gb + (i >> 3) * _FP
                    sub_a = pl.multiple_of((i & 7) * _D, _D)
                    accs = [a0, a1, a2, a3]
                    for jq in range(_QB):
                        blk_b = gb + jq * _FP + i
                        for j8 in range(8):
                            j = 8 * jq + j8
                            if j >= _F:
                                break
                            a = rows_v[blk_a + j, pl.ds(sub_a, _D)]
                            b = rows_v[blk_b, pl.ds(j8 * _D, _D)]
                            accs[j % 4] = accs[j % 4] + a * b
                    dv = rows_v[blk_a + i, pl.ds(sub_a, _D)]
                    return (accs[0], accs[1], accs[2], accs[3],
                            dg + dv * dv)

                a0, a1, a2, a3, dg = lax.fori_loop(
                    0, _F, outer, (zero, zero, zero, zero, zero))
                acc = ((a0 + a1) + (a2 + a3) - dg) * 0.5

                def lin(f, acc):
                    return acc + rows_v[gb + 3 * _FP + f,
                                        pl.ds((_LIN_SLOT % 8) * _D, _D)]

                acc = lax.fori_loop(0, _F, lin, acc)
                acc = acc + rows_v[gb + 3 * _FP,
                                   pl.ds((_BIAS_SLOT % 8) * _D, _D)]
                zvec = _lane_sum(acc)
                res = jnp.where(lanes == u * _CB + bl, zvec, res)
            return res

        res = lax.fori_loop(0, _CPG, chunk, jnp.zeros((_D,), jnp.float32))
        out_v[pl.ds(pl.multiple_of(g * _D, _D), _D)] = (
            1.0 / (1.0 + jnp.exp(-res)))
        return carry

    lax.fori_loop(0, _BPW // _D, group, 0)
    pltpu.sync_copy(out_v, out_hbm.at[pl.ds(b0, _BPW)])


def kernel(x, tables, lin_w, lin_b):
    xi = x.astype(jnp.int32)
    x32 = jnp.concatenate([xi, xi[:, :_FP - _F]], axis=1)
    tp = jnp.transpose(tables, (1, 0, 2)).reshape(_V, _F * _D)
    lin_col = jnp.pad(lin_w, ((0, 0), (0, _D - 1)))
    bias_col = jnp.pad(
        jnp.broadcast_to(lin_b.reshape(1, 1), (_V, 1)),
        ((0, 0), (0, _D - 1)))
    zpad = jnp.zeros((_V, (_SLOTS - _F - 2) * _D), jnp.float32)
    t2 = jnp.concatenate([tp, lin_col, bias_col, zpad], axis=1)
    t2 = t2.reshape(_V * _QB, 128)
    return _ffm_sc(x32.reshape(-1), t2)
